# baseline (speedup 1.0000x reference)
import functools

import jax
import jax.numpy as jnp
from jax.experimental import pallas as pl
from jax.experimental.pallas import tpu as pltpu

K = 2048
D = 256
BLK = 2048
EX = 128


def _vq_block_kernel(z_ref, cb_ref, q_ref, ids_ref, loss_ref,
                     c2_ref, rhs_ref):
    @pl.when(pl.program_id(0) == 0)
    def _prologue():
        cb0 = cb_ref[...]
        c2_ref[...] = jnp.sum(cb0 * cb0, axis=1)[None, :]
        kio = jax.lax.broadcasted_iota(jnp.int32, (K, EX), 0)
        cio = jax.lax.broadcasted_iota(jnp.int32, (K, EX), 1)
        ones_col = jnp.where(cio == 0, 1.0, 0.0)
        hi_col = jnp.where(cio == 1, ((kio // 256) * 256).astype(jnp.float32), 0.0)
        lo_col = jnp.where(cio == 2, (kio % 256).astype(jnp.float32), 0.0)
        rhs_ref[:, :D] = cb0.astype(jnp.bfloat16)
        rhs_ref[:, D:] = (ones_col + hi_col + lo_col).astype(jnp.bfloat16)

    z = z_ref[...]
    cb = cb_ref[...]

    mm = jax.lax.dot_general(
        z, cb, (((1,), (1,)), ((), ())),
        preferred_element_type=jnp.float32,
    )
    z2 = jnp.sum(z * z, axis=1, keepdims=True)
    dist = (z2 - 2.0 * mm) + c2_ref[...]

    minval = jnp.min(dist, axis=1, keepdims=True)
    mask = jnp.where(dist == minval, 1.0, 0.0).astype(jnp.bfloat16)

    res = jax.lax.dot_general(mask, rhs_ref[...], (((1,), (0,)), ((), ())),
                              preferred_element_type=jnp.float32)
    q = res[:, :D]
    cnt = res[:, D:D + 1]
    ids_f = res[:, D + 1:D + 2] + res[:, D + 2:D + 3]
    ids = ids_f.astype(jnp.int32)

    q_ref[...] = q
    ids_ref[...] = ids.reshape(1, 1, BLK)
    loss_ref[...] = (q - z) ** 2

    @pl.when(jnp.any(cnt != 1.0))
    def _tie_fallback():
        iota = jax.lax.broadcasted_iota(jnp.int32, (BLK, K), 1)
        ids_x = jnp.min(jnp.where(dist == minval, iota, K), axis=1)
        onehot = jnp.where(iota == ids_x[:, None], 1.0, 0.0)
        q_x = jax.lax.dot_general(onehot, cb, (((1,), (0,)), ((), ())),
                                  preferred_element_type=jnp.float32)
        q_ref[...] = q_x
        ids_ref[...] = ids_x.reshape(1, 1, BLK)
        loss_ref[...] = (q_x - z) ** 2


@functools.partial(jax.jit, static_argnames=())
def kernel(input, codebook):
    B, T, _ = input.shape
    n_tok = B * T
    n_blk = n_tok // BLK
    z = input.reshape(n_tok, D)

    q, ids3, loss = pl.pallas_call(
        _vq_block_kernel,
        grid=(n_blk,),
        in_specs=[
            pl.BlockSpec((BLK, D), lambda i: (i, 0)),
            pl.BlockSpec((K, D), lambda i: (0, 0)),
        ],
        out_specs=[
            pl.BlockSpec((BLK, D), lambda i: (i, 0)),
            pl.BlockSpec((1, 1, BLK), lambda i: (i, 0, 0)),
            pl.BlockSpec((BLK, D), lambda i: (i, 0)),
        ],
        out_shape=[
            jax.ShapeDtypeStruct((n_tok, D), jnp.float32),
            jax.ShapeDtypeStruct((n_blk, 1, BLK), jnp.int32),
            jax.ShapeDtypeStruct((n_tok, D), jnp.float32),
        ],
        scratch_shapes=[
            pltpu.VMEM((1, K), jnp.float32),
            pltpu.VMEM((K, D + EX), jnp.bfloat16),
        ],
    )(z, codebook)

    q = q.reshape(B, T, D)
    ids = ids3.reshape(B, T)
    loss = loss.reshape(B, T, D)
    return (q, ids, loss, loss)


# speedup vs baseline: 1.1437x; 1.1437x over previous
import functools

import jax
import jax.numpy as jnp
from jax.experimental import pallas as pl
from jax.experimental.pallas import tpu as pltpu

K = 2048
D = 256
BLK = 2048
EX = 128


def _vq_block_kernel(z_ref, cb_ref, q_ref, ids_ref, loss_ref,
                     c2_ref, rhs_ref):
    @pl.when(pl.program_id(0) == 0)
    def _prologue():
        cb0 = cb_ref[...]
        c2_ref[...] = jnp.sum(cb0 * cb0, axis=1)[None, :]
        kio = jax.lax.broadcasted_iota(jnp.int32, (K, EX), 0)
        cio = jax.lax.broadcasted_iota(jnp.int32, (K, EX), 1)
        ones_col = jnp.where(cio == 0, 1.0, 0.0)
        hi_col = jnp.where(cio == 1, ((kio // 256) * 256).astype(jnp.float32), 0.0)
        lo_col = jnp.where(cio == 2, (kio % 256).astype(jnp.float32), 0.0)
        rhs_ref[:, :D] = cb0
        rhs_ref[:, D:] = ones_col + hi_col + lo_col

    z = z_ref[...]
    cb = cb_ref[...]

    mm = jax.lax.dot_general(
        z, cb, (((1,), (1,)), ((), ())),
        preferred_element_type=jnp.float32,
    )
    z2 = jnp.sum(z * z, axis=1, keepdims=True)
    dist = (z2 - 2.0 * mm) + c2_ref[...]

    minval = jnp.min(dist, axis=1, keepdims=True)
    mask = jnp.where(dist == minval, 1.0, 0.0)

    res = jax.lax.dot_general(mask, rhs_ref[...], (((1,), (0,)), ((), ())),
                              preferred_element_type=jnp.float32)
    q = res[:, :D]
    cnt = res[:, D:D + 1]
    ids_f = res[:, D + 1:D + 2] + res[:, D + 2:D + 3]
    ids = ids_f.astype(jnp.int32)

    q_ref[...] = q
    ids_ref[...] = ids.reshape(1, 1, BLK)
    loss_ref[...] = (q - z) ** 2

    @pl.when(jnp.any(cnt != 1.0))
    def _tie_fallback():
        iota = jax.lax.broadcasted_iota(jnp.int32, (BLK, K), 1)
        ids_x = jnp.min(jnp.where(dist == minval, iota, K), axis=1)
        onehot = jnp.where(iota == ids_x[:, None], 1.0, 0.0)
        q_x = jax.lax.dot_general(onehot, cb, (((1,), (0,)), ((), ())),
                                  preferred_element_type=jnp.float32)
        q_ref[...] = q_x
        ids_ref[...] = ids_x.reshape(1, 1, BLK)
        loss_ref[...] = (q_x - z) ** 2


@functools.partial(jax.jit, static_argnames=())
def kernel(input, codebook):
    B, T, _ = input.shape
    n_tok = B * T
    n_blk = n_tok // BLK
    z = input.reshape(n_tok, D)

    q, ids3, loss = pl.pallas_call(
        _vq_block_kernel,
        grid=(n_blk,),
        in_specs=[
            pl.BlockSpec((BLK, D), lambda i: (i, 0)),
            pl.BlockSpec((K, D), lambda i: (0, 0)),
        ],
        out_specs=[
            pl.BlockSpec((BLK, D), lambda i: (i, 0)),
            pl.BlockSpec((1, 1, BLK), lambda i: (i, 0, 0)),
            pl.BlockSpec((BLK, D), lambda i: (i, 0)),
        ],
        out_shape=[
            jax.ShapeDtypeStruct((n_tok, D), jnp.float32),
            jax.ShapeDtypeStruct((n_blk, 1, BLK), jnp.int32),
            jax.ShapeDtypeStruct((n_tok, D), jnp.float32),
        ],
        scratch_shapes=[
            pltpu.VMEM((1, K), jnp.float32),
            pltpu.VMEM((K, D + EX), jnp.float32),
        ],
    )(z, codebook)

    q = q.reshape(B, T, D)
    ids = ids3.reshape(B, T)
    loss = loss.reshape(B, T, D)
    return (q, ids, loss, loss)


# speedup vs baseline: 1.1591x; 1.0134x over previous
import functools

import jax
import jax.numpy as jnp
from jax.experimental import pallas as pl
from jax.experimental.pallas import tpu as pltpu

K = 2048
D = 256
BLK = 1024
EX = 128


def _vq_block_kernel(z_ref, cb_ref, q_ref, ids_ref, loss_ref,
                     c2_ref, rhs_ref):
    @pl.when(pl.program_id(0) == 0)
    def _prologue():
        cb0 = cb_ref[...]
        c2_ref[...] = jnp.sum(cb0 * cb0, axis=1)[None, :]
        kio = jax.lax.broadcasted_iota(jnp.int32, (K, EX), 0)
        cio = jax.lax.broadcasted_iota(jnp.int32, (K, EX), 1)
        ones_col = jnp.where(cio == 0, 1.0, 0.0)
        hi_col = jnp.where(cio == 1, ((kio // 256) * 256).astype(jnp.float32), 0.0)
        lo_col = jnp.where(cio == 2, (kio % 256).astype(jnp.float32), 0.0)
        rhs_ref[:, :D] = cb0
        rhs_ref[:, D:] = ones_col + hi_col + lo_col

    z = z_ref[...]
    cb = cb_ref[...]

    mm = jax.lax.dot_general(
        z, cb, (((1,), (1,)), ((), ())),
        preferred_element_type=jnp.float32,
    )
    z2 = jnp.sum(z * z, axis=1, keepdims=True)
    dist = (z2 - 2.0 * mm) + c2_ref[...]

    minval = jnp.min(dist, axis=1, keepdims=True)
    mask = jnp.where(dist == minval, 1.0, 0.0)

    res = jax.lax.dot_general(mask, rhs_ref[...], (((1,), (0,)), ((), ())),
                              preferred_element_type=jnp.float32)
    q = res[:, :D]
    cnt = res[:, D:D + 1]
    ids_f = res[:, D + 1:D + 2] + res[:, D + 2:D + 3]
    ids = ids_f.astype(jnp.int32)

    q_ref[...] = q
    ids_ref[...] = ids.reshape(1, 1, BLK)
    loss_ref[...] = (q - z) ** 2

    @pl.when(jnp.any(cnt != 1.0))
    def _tie_fallback():
        iota = jax.lax.broadcasted_iota(jnp.int32, (BLK, K), 1)
        ids_x = jnp.min(jnp.where(dist == minval, iota, K), axis=1)
        onehot = jnp.where(iota == ids_x[:, None], 1.0, 0.0)
        q_x = jax.lax.dot_general(onehot, cb, (((1,), (0,)), ((), ())),
                                  preferred_element_type=jnp.float32)
        q_ref[...] = q_x
        ids_ref[...] = ids_x.reshape(1, 1, BLK)
        loss_ref[...] = (q_x - z) ** 2


@functools.partial(jax.jit, static_argnames=())
def kernel(input, codebook):
    B, T, _ = input.shape
    n_tok = B * T
    n_blk = n_tok // BLK
    z = input.reshape(n_tok, D)

    q, ids3, loss = pl.pallas_call(
        _vq_block_kernel,
        grid=(n_blk,),
        in_specs=[
            pl.BlockSpec((BLK, D), lambda i: (i, 0)),
            pl.BlockSpec((K, D), lambda i: (0, 0)),
        ],
        out_specs=[
            pl.BlockSpec((BLK, D), lambda i: (i, 0)),
            pl.BlockSpec((1, 1, BLK), lambda i: (i, 0, 0)),
            pl.BlockSpec((BLK, D), lambda i: (i, 0)),
        ],
        out_shape=[
            jax.ShapeDtypeStruct((n_tok, D), jnp.float32),
            jax.ShapeDtypeStruct((n_blk, 1, BLK), jnp.int32),
            jax.ShapeDtypeStruct((n_tok, D), jnp.float32),
        ],
        scratch_shapes=[
            pltpu.VMEM((1, K), jnp.float32),
            pltpu.VMEM((K, D + EX), jnp.float32),
        ],
    )(z, codebook)

    q = q.reshape(B, T, D)
    ids = ids3.reshape(B, T)
    loss = loss.reshape(B, T, D)
    return (q, ids, loss, loss)
